# baseline (speedup 1.0000x reference)
import jax
import jax.numpy as jnp
from jax import lax
from jax.experimental import pallas as pl
from jax.experimental.pallas import tpu as pltpu
from jax.experimental.pallas import tpu_sc as plsc
from jax.experimental.layout import Layout, with_layout_constraint

NUM_FACTORS = 64
BATCH = 16384
NC, NS, L = 2, 16, 16
NW = NC * NS
B_PER_W = BATCH // NW
CHUNK = 256
N_CHUNKS = B_PER_W // CHUNK
NK = NUM_FACTORS // L


def _body(user_idx, item_i_idx, feature_i_idx, item_j_idx, feature_j_idx,
          user_table, item_table, visual_table, visual_bias_table,
          out_hbm,
          iu, ii, ifi, ij, ifj,
          u_buf, ti_buf, tj_buf, vi_buf, vj_buf, bi_buf, bj_buf,
          out_buf, bsem, rsem):
    wid = lax.axis_index("s") * NC + lax.axis_index("c")
    base = wid * B_PER_W
    lane = lax.iota(jnp.int32, L)

    for c in range(N_CHUNKS):
        off = base + c * CHUNK
        pltpu.sync_copy(user_idx.at[pl.ds(off, CHUNK)], iu)
        pltpu.sync_copy(item_i_idx.at[pl.ds(off, CHUNK)], ii)
        pltpu.sync_copy(feature_i_idx.at[pl.ds(off, CHUNK)], ifi)
        pltpu.sync_copy(item_j_idx.at[pl.ds(off, CHUNK)], ij)
        pltpu.sync_copy(feature_j_idx.at[pl.ds(off, CHUNK)], ifj)
        cps = [
            pltpu.async_copy(user_table.at[iu], u_buf, rsem),
            pltpu.async_copy(item_table.at[ii], ti_buf, rsem),
            pltpu.async_copy(item_table.at[ij], tj_buf, rsem),
            pltpu.async_copy(visual_table.at[ifi], vi_buf, rsem),
            pltpu.async_copy(visual_table.at[ifj], vj_buf, rsem),
            pltpu.async_copy(visual_bias_table.at[ifi], bi_buf, bsem),
            pltpu.async_copy(visual_bias_table.at[ifj], bj_buf, bsem),
        ]
        for cp in cps:
            cp.wait()

        def group_body(g, carry):
            def row_body(rr, acc):
                r = g * L + rr
                p = jnp.zeros((L,), jnp.float32)
                for k in range(NK):
                    sl = pl.ds(k * L, L)
                    comb = (ti_buf[r, sl] - tj_buf[r, sl]
                            + vi_buf[r, sl] - vj_buf[r, sl])
                    p = p + u_buf[r, sl] * comb
                for s in (8, 4, 2, 1):
                    p = p + p[jnp.bitwise_xor(lane, s)]
                return jnp.where(lane == rr, p + acc, acc)

            acc0 = bi_buf[pl.ds(g * L, L)] - bj_buf[pl.ds(g * L, L)]
            acc = lax.fori_loop(0, L, row_body, acc0)
            out_buf[pl.ds(c * CHUNK + g * L, L)] = acc
            return carry

        lax.fori_loop(0, CHUNK // L, group_body, 0)

    pltpu.sync_copy(out_buf, out_hbm.at[pl.ds(base, B_PER_W)])


@jax.jit
def _run(user, item_i, feature_i, item_j, feature_j,
         user_table, item_table, visual_table, visual_bias_table):
    mesh = plsc.VectorSubcoreMesh(core_axis_name="c", subcore_axis_name="s")
    grid_kernel = pl.kernel(
        _body,
        out_type=jax.ShapeDtypeStruct((BATCH,), jnp.float32),
        mesh=mesh,
        compiler_params=pltpu.CompilerParams(use_tc_tiling_on_sc=False),
        scratch_types=(
            [pltpu.VMEM((CHUNK,), jnp.int32) for _ in range(5)]
            + [pltpu.VMEM((CHUNK, NUM_FACTORS), jnp.float32)
               for _ in range(5)]
            + [pltpu.VMEM((CHUNK,), jnp.float32) for _ in range(2)]
            + [pltpu.VMEM((B_PER_W,), jnp.float32),
               pltpu.SemaphoreType.DMA,
               pltpu.SemaphoreType.DMA]
        ),
    )
    fmt = Layout(major_to_minor=(0, 1), tiling=((8,),))
    user_table, item_table, visual_table = lax.optimization_barrier(
        tuple(with_layout_constraint(t, fmt)
              for t in (user_table, item_table, visual_table)))
    return grid_kernel(user, item_i, feature_i, item_j, feature_j,
                       user_table, item_table, visual_table,
                       visual_bias_table)


def kernel(user, item_i, feature_i, item_j, feature_j,
           user_table, item_table, visual_table, visual_bias_table):
    return _run(user.astype(jnp.int32), item_i.astype(jnp.int32),
                feature_i.astype(jnp.int32), item_j.astype(jnp.int32),
                feature_j.astype(jnp.int32),
                user_table, item_table, visual_table,
                visual_bias_table.reshape(-1))


# speedup vs baseline: 2.2548x; 2.2548x over previous
import jax
import jax.numpy as jnp
from jax import lax
from jax.experimental import pallas as pl
from jax.experimental.pallas import tpu as pltpu
from jax.experimental.pallas import tpu_sc as plsc
from jax.experimental.layout import Layout, with_layout_constraint

NUM_FACTORS = 64
BATCH = 16384
NC, NS, L = 2, 16, 16
NW = NC * NS
B_PER_W = BATCH // NW
NG = B_PER_W // L
NK = NUM_FACTORS // L


def _body(user_idx, item_i_idx, feature_i_idx, item_j_idx, feature_j_idx,
          user_table, item_table, visual_table, visual_bias_table,
          out_hbm,
          iu, ii, ifi, ij, ifj,
          bu, bti, btj, bvi, bvj, bi_buf, bj_buf,
          out_buf, bsem, rsem):
    wid = lax.axis_index("s") * NC + lax.axis_index("c")
    base = wid * B_PER_W
    lane = lax.iota(jnp.int32, L)

    pltpu.sync_copy(user_idx.at[pl.ds(base, B_PER_W)], iu)
    pltpu.sync_copy(item_i_idx.at[pl.ds(base, B_PER_W)], ii)
    pltpu.sync_copy(feature_i_idx.at[pl.ds(base, B_PER_W)], ifi)
    pltpu.sync_copy(item_j_idx.at[pl.ds(base, B_PER_W)], ij)
    pltpu.sync_copy(feature_j_idx.at[pl.ds(base, B_PER_W)], ifj)
    bcps = [pltpu.async_copy(visual_bias_table.at[ifi], bi_buf, bsem),
            pltpu.async_copy(visual_bias_table.at[ifj], bj_buf, bsem)]
    for cp in bcps:
        cp.wait()

    sets = ((iu, user_table, bu), (ii, item_table, bti),
            (ij, item_table, btj), (ifi, visual_table, bvi),
            (ifj, visual_table, bvj))

    def group_body(g, carry):
        vecs = [idxb[pl.ds(g * L, L)] for idxb, _, _ in sets]
        cps = []
        for (idxb, tab, blk), vec in zip(sets, vecs):
            for jj in range(L):
                q = vec[jj]
                blk8 = pl.multiple_of(
                    lax.shift_left(lax.shift_right_logical(q, 3), 3), 8)
                cps.append(pltpu.async_copy(
                    tab.at[pl.ds(blk8, 8), :], blk.at[jj], rsem))
        for cp in cps:
            cp.wait()

        acc = bi_buf[pl.ds(g * L, L)] - bj_buf[pl.ds(g * L, L)]
        for jj in range(L):
            su = jnp.bitwise_and(vecs[0][jj], 7)
            sti = jnp.bitwise_and(vecs[1][jj], 7)
            stj = jnp.bitwise_and(vecs[2][jj], 7)
            svi = jnp.bitwise_and(vecs[3][jj], 7)
            svj = jnp.bitwise_and(vecs[4][jj], 7)
            p = jnp.zeros((L,), jnp.float32)
            for k in range(NK):
                sl = pl.ds(k * L, L)
                comb = (bti[jj, sti, sl] - btj[jj, stj, sl]
                        + bvi[jj, svi, sl] - bvj[jj, svj, sl])
                p = p + bu[jj, su, sl] * comb
            for s in (8, 4, 2, 1):
                p = p + p[jnp.bitwise_xor(lane, s)]
            acc = jnp.where(lane == jj, p + acc, acc)
        out_buf[pl.ds(g * L, L)] = acc
        return carry

    lax.fori_loop(0, NG, group_body, 0)
    pltpu.sync_copy(out_buf, out_hbm.at[pl.ds(base, B_PER_W)])


@jax.jit
def _run(user, item_i, feature_i, item_j, feature_j,
         user_table, item_table, visual_table, visual_bias_table):
    mesh = plsc.VectorSubcoreMesh(core_axis_name="c", subcore_axis_name="s")
    grid_kernel = pl.kernel(
        _body,
        out_type=jax.ShapeDtypeStruct((BATCH,), jnp.float32),
        mesh=mesh,
        scratch_types=(
            [pltpu.VMEM((B_PER_W,), jnp.int32) for _ in range(5)]
            + [pltpu.VMEM((L, 8, NUM_FACTORS), jnp.float32)
               for _ in range(5)]
            + [pltpu.VMEM((B_PER_W,), jnp.float32) for _ in range(2)]
            + [pltpu.VMEM((B_PER_W,), jnp.float32),
               pltpu.SemaphoreType.DMA,
               pltpu.SemaphoreType.DMA]
        ),
    )
    fmt = Layout(major_to_minor=(0, 1))
    user_table, item_table, visual_table = lax.optimization_barrier(
        tuple(with_layout_constraint(t, fmt)
              for t in (user_table, item_table, visual_table)))
    return grid_kernel(user, item_i, feature_i, item_j, feature_j,
                       user_table, item_table, visual_table,
                       visual_bias_table)


def kernel(user, item_i, feature_i, item_j, feature_j,
           user_table, item_table, visual_table, visual_bias_table):
    return _run(user.astype(jnp.int32), item_i.astype(jnp.int32),
                feature_i.astype(jnp.int32), item_j.astype(jnp.int32),
                feature_j.astype(jnp.int32),
                user_table, item_table, visual_table,
                visual_bias_table.reshape(-1))
